# baseline (speedup 1.0000x reference)
import functools

import jax
import jax.numpy as jnp
from jax import lax
from jax.experimental import pallas as pl
from jax.experimental.pallas import tpu as pltpu
from jax.experimental.pallas import tpu_sc as plsc

N = 16384
K = 4096
NS = 16
R2 = 0.25
D = 64
TW = 128

NC = 2
NSUB = 16
NW = NC * NSUB

_BIGF = 1e9
_INF = float("inf")


_BI = 256
_CJ = 2048


def _rank_body(scol_ref, srow_ref, rank_ref):
    si = scol_ref[...]
    bi = pl.program_id(0)
    cb = bi // (_CJ // _BI)
    i_glob = bi * _BI + lax.broadcasted_iota(jnp.int32, (_BI, 1), 0)

    def step_ge(c, acc):
        sj = srow_ref[pl.ds(c, 1), :]
        return acc + jnp.sum((sj >= si).astype(jnp.int32), axis=1,
                             keepdims=True)

    def step_gt(c, acc):
        sj = srow_ref[pl.ds(c, 1), :]
        return acc + jnp.sum((sj > si).astype(jnp.int32), axis=1,
                             keepdims=True)

    acc = lax.fori_loop(0, cb, step_ge, jnp.zeros((_BI, 1), jnp.int32))
    sj = srow_ref[pl.ds(cb, 1), :]
    j_glob = cb * _CJ + lax.broadcasted_iota(jnp.int32, (1, _CJ), 1)
    cmp = (sj > si) | ((sj == si) & (j_glob < i_glob))
    acc = acc + jnp.sum(cmp.astype(jnp.int32), axis=1, keepdims=True)
    rank_ref[...] = lax.fori_loop(cb + 1, N // _CJ, step_gt, acc)


def _ranks(score):
    scol = score.reshape(N, 1)
    srow = score.reshape(N // _CJ, _CJ)
    rank = pl.pallas_call(
        _rank_body,
        grid=(N // _BI,),
        in_specs=[
            pl.BlockSpec((_BI, 1), lambda b: (b, 0)),
            pl.BlockSpec((N // _CJ, _CJ), lambda b: (0, 0)),
        ],
        out_specs=pl.BlockSpec((_BI, 1), lambda b: (b, 0)),
        out_shape=jax.ShapeDtypeStruct((N, 1), jnp.int32),
    )(scol, srow)
    return rank.reshape(N // 128, 128)


_QROWS = K + 8


def _sc_scatter(rank2d, src):
    mesh = plsc.VectorSubcoreMesh(core_axis_name="c", subcore_axis_name="s")
    chunk = N // NW
    nsub = chunk // 128

    @functools.partial(
        pl.kernel,
        mesh=mesh,
        out_type=jax.ShapeDtypeStruct((_QROWS, TW), jnp.float32),
        scratch_types=[
            pltpu.VMEM((nsub, 128), jnp.int32),
            pltpu.VMEM((nsub, 128), jnp.int32),
            pltpu.VMEM((chunk, TW), jnp.float32),
            pltpu.SemaphoreType.DMA,
        ],
    )
    def k(rank_hbm, src_hbm, out_hbm, rank_v, tgt_v, src_v, sem):
        wid = lax.axis_index("s") * NC + lax.axis_index("c")
        pltpu.sync_copy(rank_hbm.at[pl.ds(wid * nsub, nsub)], rank_v)
        pltpu.sync_copy(src_hbm.at[pl.ds(wid * chunk, chunk)], src_v)
        for r in range(nsub):
            for c in range(8):
                rk = rank_v[r, pl.ds(c * 16, 16)]
                i0 = wid * chunk + r * 128 + c * 16
                iv = i0 + lax.iota(jnp.int32, 16)
                tgt = jnp.where(rk < K, rk, K + (iv & 7))
                tgt_v[r, pl.ds(c * 16, 16)] = tgt
        cps = []
        for r in range(nsub):
            cps.append(pltpu.async_copy(
                src_v.at[pl.ds(r * 128, 128)], out_hbm.at[tgt_v.at[r]], sem))
        for cp in cps:
            cp.wait()

    return k(rank2d, src)


_BR = 256


_NG = 128
_GL = N // _NG
_S = 3


def _radius_body(posP_ref, qT_ref, nbrIT_ref, nbrDT_ref):
    px = posP_ref[:, 0:1]
    py = posP_ref[:, 1:2]
    pz = posP_ref[:, 2:3]
    qx = qT_ref[pl.ds(0, 1), :]
    qy = qT_ref[pl.ds(1, 1), :]
    qz = qT_ref[pl.ds(2, 1), :]
    dx = px - qx
    dy = py - qy
    dz = pz - qz
    d2 = dx * dx + dy * dy + dz * dz
    g3 = d2.reshape(_NG, _GL, _BR)

    aio3 = lax.broadcasted_iota(jnp.int32, (_NG, _GL, _BR), 1).astype(
        jnp.float32)
    vs, ls = [], []
    g3m = g3
    for s in range(_S):
        mcol = jnp.min(g3m, axis=1)
        acol = jnp.min(jnp.where(g3m == mcol[:, None, :], aio3, _BIGF),
                       axis=1)
        vs.append(mcol)
        ls.append(acol)
        if s + 1 < _S:
            g3m = jnp.where(aio3 == acol[:, None, :], _INF, g3m)

    giota = lax.broadcasted_iota(jnp.int32, (_NG, _BR), 0).astype(jnp.float32)
    aio2 = lax.broadcasted_iota(jnp.int32, (_GL, _BR), 0).astype(jnp.float32)
    srow = lax.broadcasted_iota(jnp.int32, (NS, _BR), 0)

    def refill(args):
        h, hl, ghot, needre, mb, ab = args
        tcol = jnp.min(jnp.where(ghot[:, None, :], g3, _INF), axis=0)
        ok = (tcol > mb) | ((tcol == mb) & (aio2 > ab))
        t2 = jnp.where(ok, tcol, _INF)
        hnew = jnp.min(t2, axis=0, keepdims=True)
        anew = jnp.min(jnp.where(t2 == hnew, aio2, _BIGF), axis=0,
                       keepdims=True)
        sel = ghot & needre
        return jnp.where(sel, hnew, h), jnp.where(sel, anew, hl)

    def step(s, carry):
        h, hl, cnt, outD, outJ = carry
        m = jnp.min(h, axis=0, keepdims=True)
        gstar = jnp.min(jnp.where(h == m, giota, _BIGF), axis=0,
                        keepdims=True)
        ghot = giota == gstar
        active = m < _INF
        astar = jnp.min(jnp.where(ghot, hl, _BIGF), axis=0, keepdims=True)
        jstar = gstar * float(_GL) + astar
        outD = jnp.where(srow == s, m, outD)
        outJ = jnp.where(srow == s, jstar, outJ)
        upd = ghot & active
        cnt = cnt + upd.astype(jnp.int32)
        c_at = jnp.max(jnp.where(upd, cnt, 0), axis=0, keepdims=True)
        hv = _INF * jnp.ones((1, _BR), jnp.float32)
        lv = jnp.zeros((1, _BR), jnp.float32)
        for kidx in range(_S - 1, 0, -1):
            vk = jnp.min(jnp.where(ghot, vs[kidx], _INF), axis=0,
                         keepdims=True)
            lk = jnp.min(jnp.where(ghot, ls[kidx], _BIGF), axis=0,
                         keepdims=True)
            hit = c_at == kidx
            hv = jnp.where(hit, vk, hv)
            lv = jnp.where(hit, lk, lv)
        h = jnp.where(upd, hv, h)
        hl = jnp.where(upd, lv, hl)
        needre = active & (c_at >= _S)
        h, hl = lax.cond(jnp.any(needre), refill,
                         lambda args: (args[0], args[1]),
                         (h, hl, ghot, needre, m, astar))
        return h, hl, cnt, outD, outJ

    _, _, _, outD, outJ = lax.fori_loop(
        0, NS, step,
        (vs[0], ls[0], jnp.zeros((_NG, _BR), jnp.int32),
         jnp.zeros((NS, _BR), jnp.float32),
         jnp.zeros((NS, _BR), jnp.float32)))
    nbrIT_ref[...] = outJ.astype(jnp.int32)
    nbrDT_ref[...] = outD


def _radius_topk(posP, qT):
    kh = qT.shape[1]
    return pl.pallas_call(
        _radius_body,
        grid=(kh // _BR,),
        in_specs=[
            pl.BlockSpec((N, 8), lambda b: (0, 0)),
            pl.BlockSpec((8, _BR), lambda b: (0, b)),
        ],
        out_specs=[
            pl.BlockSpec((NS, _BR), lambda b: (0, b)),
            pl.BlockSpec((NS, _BR), lambda b: (0, b)),
        ],
        out_shape=[
            jax.ShapeDtypeStruct((NS, kh), jnp.int32),
            jax.ShapeDtypeStruct((NS, kh), jnp.float32),
        ],
    )(posP, qT)


_E = K * NS


def _sc_gather(table, nbr2d):
    mesh = plsc.VectorSubcoreMesh(core_axis_name="c", subcore_axis_name="s")
    ne = nbr2d.shape[0] * nbr2d.shape[1]
    rows_per_w = ne // NW
    nchunk = rows_per_w // 128

    @functools.partial(
        pl.kernel,
        mesh=mesh,
        out_type=jax.ShapeDtypeStruct((ne, TW), jnp.float32),
        scratch_types=[
            pltpu.VMEM((nchunk, 128), jnp.int32),
            pltpu.VMEM((128, TW), jnp.float32),
            pltpu.VMEM((128, TW), jnp.float32),
            pltpu.VMEM((128, TW), jnp.float32),
            pltpu.VMEM((128, TW), jnp.float32),
            pltpu.SemaphoreType.DMA,
            pltpu.SemaphoreType.DMA,
            pltpu.SemaphoreType.DMA,
            pltpu.SemaphoreType.DMA,
        ],
    )
    def k(tab_hbm, idx_hbm, out_hbm, idx_v, buf0, buf1, buf2, buf3,
          sem0, sem1, sem2, sem3):
        wid = lax.axis_index("s") * NC + lax.axis_index("c")
        pltpu.sync_copy(idx_hbm.at[pl.ds(wid * nchunk, nchunk)], idx_v)
        bufs = (buf0, buf1, buf2, buf3)
        sems = (sem0, sem1, sem2, sem3)
        cps = [None, None, None, None]
        for c in range(nchunk):
            p = c & 3
            cps[p] = pltpu.async_copy(tab_hbm.at[idx_v.at[c]], bufs[p],
                                      sems[p])
            if c >= 3:
                q = (c - 3) & 3
                cps[q].wait()
                pltpu.sync_copy(
                    bufs[q],
                    out_hbm.at[pl.ds(wid * rows_per_w + (c - 3) * 128, 128)])
        for c in range(nchunk - 3, nchunk):
            q = c & 3
            cps[q].wait()
            pltpu.sync_copy(
                bufs[q],
                out_hbm.at[pl.ds(wid * rows_per_w + c * 128, 128)])

    return k(table, nbr2d)


_B5 = 256


def _mlp_body(g_ref, qb_ref, x_ref, p_ref, nbrI_ref, nbrD_ref, ns_ref,
              w1_ref, b1_ref, w2_ref, b2_ref, w3_ref, b3_ref, out_ref,
              roff=0):
    w1 = w1_ref[...]
    b1 = b1_ref[...]
    w2 = w2_ref[...]
    b2 = b2_ref[...]
    w3 = w3_ref[...]
    b3 = b3_ref[...]
    q3 = qb_ref[:, 0:3]
    qpad = jnp.concatenate(
        [jnp.zeros((_B5, D), jnp.float32), q3,
         jnp.zeros((_B5, TW - D - 3), jnp.float32)], axis=1)
    g3 = g_ref[...].reshape(_B5, NS, TW)
    h = (g3 - qpad[:, None, :]).reshape(_B5 * NS, TW)
    h = jnp.maximum(jnp.dot(h, w1, preferred_element_type=jnp.float32) + b1,
                    0.0)
    h = jnp.maximum(jnp.dot(h, w2, preferred_element_type=jnp.float32) + b2,
                    0.0)
    msg = jnp.dot(h, w3, preferred_element_type=jnp.float32) + b3

    xs = x_ref[...]
    p3 = p_ref[:, 0:3]
    hs = jnp.concatenate(
        [xs, p3 - q3, jnp.zeros((_B5, TW - D - 3), jnp.float32)], axis=1)
    hs = jnp.maximum(jnp.dot(hs, w1, preferred_element_type=jnp.float32) + b1,
                     0.0)
    hs = jnp.maximum(jnp.dot(hs, w2, preferred_element_type=jnp.float32) + b2,
                     0.0)
    msg_s = jnp.dot(hs, w3, preferred_element_type=jnp.float32) + b3

    b = pl.program_id(0)
    erow = lax.broadcasted_iota(jnp.int32, (_B5 * NS, 1), 0)
    rglob = roff + b * _B5 + (erow >> 4)
    slot = erow & (NS - 1)
    vb = ((nbrD_ref[...] <= R2) & (nbrI_ref[...] != rglob)
          & (slot < ns_ref[0, 0].astype(jnp.int32)))
    m3 = jnp.where(vb, msg, -_INF).reshape(_B5, NS, 128)
    out_ref[...] = jnp.maximum(jnp.max(m3, axis=1), msg_s)


def _mlp_max(g, qb, x, src16, nbrI2, nbrD2, ns, w1p, b1, w2, b2, w3, b3,
             roff=0):
    kh = qb.shape[0]
    nb = kh // _B5
    return pl.pallas_call(
        functools.partial(_mlp_body, roff=roff),
        grid=(nb,),
        in_specs=[
            pl.BlockSpec((_B5 * NS, TW), lambda b: (b, 0)),
            pl.BlockSpec((_B5, TW), lambda b: (b, 0)),
            pl.BlockSpec((_B5, D), lambda b: (b, 0)),
            pl.BlockSpec((_B5, TW), lambda b: (b, 0)),
            pl.BlockSpec((_B5 * NS, 1), lambda b: (b, 0)),
            pl.BlockSpec((_B5 * NS, 1), lambda b: (b, 0)),
            pl.BlockSpec((1, 1), lambda b: (0, 0)),
            pl.BlockSpec((TW, D), lambda b: (0, 0)),
            pl.BlockSpec((1, D), lambda b: (0, 0)),
            pl.BlockSpec((D, D), lambda b: (0, 0)),
            pl.BlockSpec((1, D), lambda b: (0, 0)),
            pl.BlockSpec((D, 128), lambda b: (0, 0)),
            pl.BlockSpec((1, 128), lambda b: (0, 0)),
        ],
        out_specs=pl.BlockSpec((_B5, 128), lambda b: (b, 0)),
        out_shape=jax.ShapeDtypeStruct((kh, 128), jnp.float32),
    )(g, qb, x, src16, nbrI2, nbrD2, ns, w1p, b1, w2, b2, w3, b3)


def kernel(x, score, pos, batch, num_samples, W1, b1, W2, b2, W3, b3):
    score = score.astype(jnp.float32)
    rank2d = _ranks(score)

    ibits = lax.bitcast_convert_type(jnp.arange(N, dtype=jnp.int32),
                                     jnp.float32).reshape(N, 1)
    bbits = lax.bitcast_convert_type(batch.astype(jnp.int32),
                                     jnp.float32).reshape(N, 1)
    src16 = jnp.concatenate(
        [pos, ibits, bbits, jnp.zeros((N, TW - 5), jnp.float32)], axis=1)

    qbuf = _sc_scatter(rank2d, src16)
    qb = qbuf[:K]

    posP = jnp.concatenate([pos, jnp.zeros((N, 5), jnp.float32)], axis=1)
    qT = jnp.concatenate([qbuf[:K, 0:3].T, jnp.zeros((5, K), jnp.float32)],
                         axis=0)
    table = jnp.concatenate(
        [x, pos, jnp.zeros((N, TW - D - 3), jnp.float32)], axis=1)
    ns = jnp.asarray(num_samples).astype(jnp.float32).reshape(1, 1)
    w1p = jnp.concatenate([W1, jnp.zeros((TW - (D + 3), D), jnp.float32)],
                          axis=0)

    kh = K // 2
    eh = kh * NS
    outs = []
    for off in (0, kh):
        nbrITh, nbrDTh = _radius_topk(posP, qT[:, off:off + kh])
        nbrIh = nbrITh.T
        nbrDh = nbrDTh.T
        gh = _sc_gather(table, nbrIh.reshape(eh // 128, 128))
        outs.append(_mlp_max(
            gh, qb[off:off + kh], x[off:off + kh], src16[off:off + kh],
            nbrIh.reshape(eh, 1), nbrDh.reshape(eh, 1), ns, w1p,
            b1.reshape(1, D), W2, b2.reshape(1, D), W3, b3.reshape(1, 128),
            roff=off))
    out = jnp.concatenate(outs, axis=0)

    q_out = qbuf[:K, 0:3]
    batch_out = lax.bitcast_convert_type(qbuf[:K, 4], jnp.int32)
    return out, q_out, batch_out


# speedup vs baseline: 1.0257x; 1.0257x over previous
import functools

import jax
import jax.numpy as jnp
from jax import lax
from jax.experimental import pallas as pl
from jax.experimental.pallas import tpu as pltpu
from jax.experimental.pallas import tpu_sc as plsc

N = 16384
K = 4096
NS = 16
R2 = 0.25
D = 64
TW = 128

NC = 2
NSUB = 16
NW = NC * NSUB

_BIGF = 1e9
_INF = float("inf")


_BI = 256
_CJ = 2048


def _rank_body(scol_ref, srow_ref, rank_ref):
    si = scol_ref[...]
    bi = pl.program_id(0)
    cb = bi // (_CJ // _BI)
    i_glob = bi * _BI + lax.broadcasted_iota(jnp.int32, (_BI, 1), 0)

    def step_ge(c, acc):
        sj = srow_ref[pl.ds(c, 1), :]
        return acc + jnp.sum((sj >= si).astype(jnp.int32), axis=1,
                             keepdims=True)

    def step_gt(c, acc):
        sj = srow_ref[pl.ds(c, 1), :]
        return acc + jnp.sum((sj > si).astype(jnp.int32), axis=1,
                             keepdims=True)

    acc = lax.fori_loop(0, cb, step_ge, jnp.zeros((_BI, 1), jnp.int32))
    sj = srow_ref[pl.ds(cb, 1), :]
    j_glob = cb * _CJ + lax.broadcasted_iota(jnp.int32, (1, _CJ), 1)
    cmp = (sj > si) | ((sj == si) & (j_glob < i_glob))
    acc = acc + jnp.sum(cmp.astype(jnp.int32), axis=1, keepdims=True)
    rank_ref[...] = lax.fori_loop(cb + 1, N // _CJ, step_gt, acc)


def _ranks(score):
    scol = score.reshape(N, 1)
    srow = score.reshape(N // _CJ, _CJ)
    rank = pl.pallas_call(
        _rank_body,
        grid=(N // _BI,),
        in_specs=[
            pl.BlockSpec((_BI, 1), lambda b: (b, 0)),
            pl.BlockSpec((N // _CJ, _CJ), lambda b: (0, 0)),
        ],
        out_specs=pl.BlockSpec((_BI, 1), lambda b: (b, 0)),
        out_shape=jax.ShapeDtypeStruct((N, 1), jnp.int32),
    )(scol, srow)
    return rank.reshape(N // 128, 128)


_QROWS = K + 8


def _sc_scatter(rank2d, src):
    mesh = plsc.VectorSubcoreMesh(core_axis_name="c", subcore_axis_name="s")
    chunk = N // NW
    nsub = chunk // 128

    @functools.partial(
        pl.kernel,
        mesh=mesh,
        out_type=jax.ShapeDtypeStruct((_QROWS, TW), jnp.float32),
        scratch_types=[
            pltpu.VMEM((nsub, 128), jnp.int32),
            pltpu.VMEM((nsub, 128), jnp.int32),
            pltpu.VMEM((chunk, TW), jnp.float32),
            pltpu.SemaphoreType.DMA,
        ],
    )
    def k(rank_hbm, src_hbm, out_hbm, rank_v, tgt_v, src_v, sem):
        wid = lax.axis_index("s") * NC + lax.axis_index("c")
        pltpu.sync_copy(rank_hbm.at[pl.ds(wid * nsub, nsub)], rank_v)
        pltpu.sync_copy(src_hbm.at[pl.ds(wid * chunk, chunk)], src_v)
        for r in range(nsub):
            for c in range(8):
                rk = rank_v[r, pl.ds(c * 16, 16)]
                i0 = wid * chunk + r * 128 + c * 16
                iv = i0 + lax.iota(jnp.int32, 16)
                tgt = jnp.where(rk < K, rk, K + (iv & 7))
                tgt_v[r, pl.ds(c * 16, 16)] = tgt
        cps = []
        for r in range(nsub):
            cps.append(pltpu.async_copy(
                src_v.at[pl.ds(r * 128, 128)], out_hbm.at[tgt_v.at[r]], sem))
        for cp in cps:
            cp.wait()

    return k(rank2d, src)


_BR = 256


_NG = 128
_GL = N // _NG
_S = 3


def _radius_body(posP_ref, qT_ref, nbrIT_ref, nbrDT_ref):
    px = posP_ref[:, 0:1]
    py = posP_ref[:, 1:2]
    pz = posP_ref[:, 2:3]
    qx = qT_ref[pl.ds(0, 1), :]
    qy = qT_ref[pl.ds(1, 1), :]
    qz = qT_ref[pl.ds(2, 1), :]
    dx = px - qx
    dy = py - qy
    dz = pz - qz
    d2 = dx * dx + dy * dy + dz * dz
    g3 = d2.reshape(_NG, _GL, _BR)

    aio3 = lax.broadcasted_iota(jnp.int32, (_NG, _GL, _BR), 1).astype(
        jnp.float32)
    vs, ls = [], []
    g3m = g3
    for s in range(_S):
        mcol = jnp.min(g3m, axis=1)
        acol = jnp.min(jnp.where(g3m == mcol[:, None, :], aio3, _BIGF),
                       axis=1)
        vs.append(mcol)
        ls.append(acol)
        if s + 1 < _S:
            g3m = jnp.where(aio3 == acol[:, None, :], _INF, g3m)

    giota = lax.broadcasted_iota(jnp.int32, (_NG, _BR), 0).astype(jnp.float32)
    aio2 = lax.broadcasted_iota(jnp.int32, (_GL, _BR), 0).astype(jnp.float32)
    srow = lax.broadcasted_iota(jnp.int32, (NS, _BR), 0)

    def refill(args):
        h, hl, ghot, needre, mb, ab = args
        tcol = jnp.min(jnp.where(ghot[:, None, :], g3, _INF), axis=0)
        ok = (tcol > mb) | ((tcol == mb) & (aio2 > ab))
        t2 = jnp.where(ok, tcol, _INF)
        hnew = jnp.min(t2, axis=0, keepdims=True)
        anew = jnp.min(jnp.where(t2 == hnew, aio2, _BIGF), axis=0,
                       keepdims=True)
        sel = ghot & needre
        return jnp.where(sel, hnew, h), jnp.where(sel, anew, hl)

    def step(s, carry):
        h, hl, cnt, outD, outJ = carry
        m = jnp.min(h, axis=0, keepdims=True)
        gstar = jnp.min(jnp.where(h == m, giota, _BIGF), axis=0,
                        keepdims=True)
        ghot = giota == gstar
        active = m < _INF
        astar = jnp.min(jnp.where(ghot, hl, _BIGF), axis=0, keepdims=True)
        jstar = gstar * float(_GL) + astar
        outD = jnp.where(srow == s, m, outD)
        outJ = jnp.where(srow == s, jstar, outJ)
        upd = ghot & active
        cnt = cnt + upd.astype(jnp.int32)
        c_at = jnp.max(jnp.where(upd, cnt, 0), axis=0, keepdims=True)
        hv = _INF * jnp.ones((1, _BR), jnp.float32)
        lv = jnp.zeros((1, _BR), jnp.float32)
        for kidx in range(_S - 1, 0, -1):
            vk = jnp.min(jnp.where(ghot, vs[kidx], _INF), axis=0,
                         keepdims=True)
            lk = jnp.min(jnp.where(ghot, ls[kidx], _BIGF), axis=0,
                         keepdims=True)
            hit = c_at == kidx
            hv = jnp.where(hit, vk, hv)
            lv = jnp.where(hit, lk, lv)
        h = jnp.where(upd, hv, h)
        hl = jnp.where(upd, lv, hl)
        needre = active & (c_at >= _S)
        h, hl = lax.cond(jnp.any(needre), refill,
                         lambda args: (args[0], args[1]),
                         (h, hl, ghot, needre, m, astar))
        return h, hl, cnt, outD, outJ

    _, _, _, outD, outJ = lax.fori_loop(
        0, NS, step,
        (vs[0], ls[0], jnp.zeros((_NG, _BR), jnp.int32),
         jnp.zeros((NS, _BR), jnp.float32),
         jnp.zeros((NS, _BR), jnp.float32)))
    nbrIT_ref[...] = outJ.astype(jnp.int32)
    nbrDT_ref[...] = outD


def _radius_topk(posP, qT):
    return pl.pallas_call(
        _radius_body,
        grid=(K // _BR,),
        in_specs=[
            pl.BlockSpec((N, 8), lambda b: (0, 0)),
            pl.BlockSpec((8, _BR), lambda b: (0, b)),
        ],
        out_specs=[
            pl.BlockSpec((NS, _BR), lambda b: (0, b)),
            pl.BlockSpec((NS, _BR), lambda b: (0, b)),
        ],
        out_shape=[
            jax.ShapeDtypeStruct((NS, K), jnp.int32),
            jax.ShapeDtypeStruct((NS, K), jnp.float32),
        ],
    )(posP, qT)


_E = K * NS


def _sc_gather(table, nbr2d):
    mesh = plsc.VectorSubcoreMesh(core_axis_name="c", subcore_axis_name="s")
    rows_per_w = _E // NW
    nchunk = rows_per_w // 128

    @functools.partial(
        pl.kernel,
        mesh=mesh,
        out_type=jax.ShapeDtypeStruct((_E, TW), jnp.float32),
        scratch_types=[
            pltpu.VMEM((nchunk, 128), jnp.int32),
            pltpu.VMEM((128, TW), jnp.float32),
            pltpu.VMEM((128, TW), jnp.float32),
            pltpu.VMEM((128, TW), jnp.float32),
            pltpu.VMEM((128, TW), jnp.float32),
            pltpu.SemaphoreType.DMA,
            pltpu.SemaphoreType.DMA,
            pltpu.SemaphoreType.DMA,
            pltpu.SemaphoreType.DMA,
        ],
    )
    def k(tab_hbm, idx_hbm, out_hbm, idx_v, buf0, buf1, buf2, buf3,
          sem0, sem1, sem2, sem3):
        wid = lax.axis_index("s") * NC + lax.axis_index("c")
        pltpu.sync_copy(idx_hbm.at[pl.ds(wid * nchunk, nchunk)], idx_v)
        bufs = (buf0, buf1, buf2, buf3)
        sems = (sem0, sem1, sem2, sem3)
        cps = [None, None, None, None]
        for c in range(nchunk):
            p = c & 3
            cps[p] = pltpu.async_copy(tab_hbm.at[idx_v.at[c]], bufs[p],
                                      sems[p])
            if c >= 3:
                q = (c - 3) & 3
                cps[q].wait()
                pltpu.sync_copy(
                    bufs[q],
                    out_hbm.at[pl.ds(wid * rows_per_w + (c - 3) * 128, 128)])
        for c in range(nchunk - 3, nchunk):
            q = c & 3
            cps[q].wait()
            pltpu.sync_copy(
                bufs[q],
                out_hbm.at[pl.ds(wid * rows_per_w + c * 128, 128)])

    return k(table, nbr2d)


_B5 = 256


def _mlp_body(g_ref, qb_ref, x_ref, p_ref, nbrI_ref, nbrD_ref, ns_ref,
              w1_ref, b1_ref, w2_ref, b2_ref, w3_ref, b3_ref, out_ref):
    w1 = w1_ref[...]
    b1 = b1_ref[...]
    w2 = w2_ref[...]
    b2 = b2_ref[...]
    w3 = w3_ref[...]
    b3 = b3_ref[...]
    q3 = qb_ref[:, 0:3]
    qpad = jnp.concatenate(
        [jnp.zeros((_B5, D), jnp.float32), q3,
         jnp.zeros((_B5, TW - D - 3), jnp.float32)], axis=1)
    g3 = g_ref[...].reshape(_B5, NS, TW)
    h = (g3 - qpad[:, None, :]).reshape(_B5 * NS, TW)
    h = jnp.maximum(jnp.dot(h, w1, preferred_element_type=jnp.float32) + b1,
                    0.0)
    h = jnp.maximum(jnp.dot(h, w2, preferred_element_type=jnp.float32) + b2,
                    0.0)
    msg = jnp.dot(h, w3, preferred_element_type=jnp.float32) + b3

    xs = x_ref[...]
    p3 = p_ref[:, 0:3]
    hs = jnp.concatenate(
        [xs, p3 - q3, jnp.zeros((_B5, TW - D - 3), jnp.float32)], axis=1)
    hs = jnp.maximum(jnp.dot(hs, w1, preferred_element_type=jnp.float32) + b1,
                     0.0)
    hs = jnp.maximum(jnp.dot(hs, w2, preferred_element_type=jnp.float32) + b2,
                     0.0)
    msg_s = jnp.dot(hs, w3, preferred_element_type=jnp.float32) + b3

    b = pl.program_id(0)
    erow = lax.broadcasted_iota(jnp.int32, (_B5 * NS, 1), 0)
    rglob = b * _B5 + (erow >> 4)
    slot = erow & (NS - 1)
    vb = ((nbrD_ref[...] <= R2) & (nbrI_ref[...] != rglob)
          & (slot < ns_ref[0, 0].astype(jnp.int32)))
    m3 = jnp.where(vb, msg, -_INF).reshape(_B5, NS, 128)
    out_ref[...] = jnp.maximum(jnp.max(m3, axis=1), msg_s)


def _mlp_max(g, qb, x, src16, nbrI2, nbrD2, ns, w1p, b1, w2, b2, w3, b3):
    nb = K // _B5
    return pl.pallas_call(
        _mlp_body,
        grid=(nb,),
        in_specs=[
            pl.BlockSpec((_B5 * NS, TW), lambda b: (b, 0)),
            pl.BlockSpec((_B5, TW), lambda b: (b, 0)),
            pl.BlockSpec((_B5, D), lambda b: (b, 0)),
            pl.BlockSpec((_B5, TW), lambda b: (b, 0)),
            pl.BlockSpec((_B5 * NS, 1), lambda b: (b, 0)),
            pl.BlockSpec((_B5 * NS, 1), lambda b: (b, 0)),
            pl.BlockSpec((1, 1), lambda b: (0, 0)),
            pl.BlockSpec((TW, D), lambda b: (0, 0)),
            pl.BlockSpec((1, D), lambda b: (0, 0)),
            pl.BlockSpec((D, D), lambda b: (0, 0)),
            pl.BlockSpec((1, D), lambda b: (0, 0)),
            pl.BlockSpec((D, 128), lambda b: (0, 0)),
            pl.BlockSpec((1, 128), lambda b: (0, 0)),
        ],
        out_specs=pl.BlockSpec((_B5, 128), lambda b: (b, 0)),
        out_shape=jax.ShapeDtypeStruct((K, 128), jnp.float32),
    )(g, qb, x, src16, nbrI2, nbrD2, ns, w1p, b1, w2, b2, w3, b3)


def kernel(x, score, pos, batch, num_samples, W1, b1, W2, b2, W3, b3):
    score = score.astype(jnp.float32)
    rank2d = _ranks(score)

    ibits = lax.bitcast_convert_type(jnp.arange(N, dtype=jnp.int32),
                                     jnp.float32).reshape(N, 1)
    bbits = lax.bitcast_convert_type(batch.astype(jnp.int32),
                                     jnp.float32).reshape(N, 1)
    src16 = jnp.concatenate(
        [pos, ibits, bbits, jnp.zeros((N, TW - 5), jnp.float32)], axis=1)

    qbuf = _sc_scatter(rank2d, src16)
    qb = qbuf[:K]

    posP = jnp.concatenate([pos, jnp.zeros((N, 5), jnp.float32)], axis=1)
    qT = jnp.concatenate([qbuf[:K, 0:3].T, jnp.zeros((5, K), jnp.float32)],
                         axis=0)
    nbrIT, nbrDT = _radius_topk(posP, qT)
    nbrI = nbrIT.T
    nbrD = nbrDT.T

    table = jnp.concatenate(
        [x, pos, jnp.zeros((N, TW - D - 3), jnp.float32)], axis=1)
    g = _sc_gather(table, nbrI.reshape(_E // 128, 128))

    ns = jnp.asarray(num_samples).astype(jnp.float32).reshape(1, 1)
    w1p = jnp.concatenate([W1, jnp.zeros((TW - (D + 3), D), jnp.float32)],
                          axis=0)
    out = _mlp_max(g, qb, x, src16, nbrI.reshape(_E, 1), nbrD.reshape(_E, 1),
                   ns, w1p, b1.reshape(1, D), W2, b2.reshape(1, D), W3,
                   b3.reshape(1, 128))

    q_out = qbuf[:K, 0:3]
    batch_out = lax.bitcast_convert_type(qbuf[:K, 4], jnp.int32)
    return out, q_out, batch_out
